# baseline (speedup 1.0000x reference)
import functools

import jax
import jax.numpy as jnp
from jax import lax
from jax.experimental import pallas as pl
from jax.experimental.pallas import tpu as pltpu
from jax.experimental.pallas import tpu_sc as plsc


_KC = 256


def _dist_argmin_body(x_ref, e_ref, idx_ref):
    xn = jnp.transpose(x_ref[0])
    ew = e_ref[...]
    a2 = jnp.sum(xn * xn, axis=1, keepdims=True)
    b2 = jnp.sum(ew * ew, axis=1)[None, :]
    K = ew.shape[0]
    m = None
    idx = None
    for c in range(0, K, _KC):
        ab = lax.dot_general(xn, ew[c:c + _KC],
                             (((1,), (1,)), ((), ())),
                             preferred_element_type=jnp.float32)
        d2 = jnp.maximum(a2 + b2[:, c:c + _KC] - 2.0 * ab, 0.0)
        d = jnp.sqrt(d2)
        mc = jnp.min(d, axis=1, keepdims=True)
        ic = jnp.argmin(d, axis=1).astype(jnp.int32) + c
        if m is None:
            m, idx = mc, ic
        else:
            take = mc < m
            m = jnp.where(take, mc, m)
            idx = jnp.where(take[:, 0], ic, idx)
    idx_ref[...] = idx.reshape(idx_ref.shape)


def _st_loss_body(x_ref, q_ref, st_ref, loss_ref):
    xv = jnp.transpose(x_ref[0])
    qv = q_ref[...]
    diff = qv - xv
    st_ref[...] = xv + diff

    @pl.when(pl.program_id(0) == 0)
    def _init():
        loss_ref[...] = jnp.zeros((1, 1), jnp.float32)

    loss_ref[...] += jnp.sum(diff * diff).reshape(1, 1)


def _make_sc_gather(K, D, NTOT):
    info = plsc.get_sparse_core_info()
    nw = info.num_cores * info.num_subcores
    rows_per_w = NTOT // nw
    n_chunks = rows_per_w // 128
    mesh = plsc.VectorSubcoreMesh(core_axis_name="c", subcore_axis_name="s")

    @functools.partial(
        pl.kernel, mesh=mesh,
        out_type=jax.ShapeDtypeStruct((NTOT, D), jnp.float32),
        scratch_types=[
            pltpu.VMEM((n_chunks, 128), jnp.int32),
            pltpu.VMEM((rows_per_w, D), jnp.float32),
            pltpu.SemaphoreType.DMA,
            pltpu.SemaphoreType.DMA,
        ],
    )
    def sc_gather(table_hbm, idx_hbm, out_hbm, idx_v, rows_v, gsem, wsem):
        wid = lax.axis_index("s") * info.num_cores + lax.axis_index("c")
        base = wid * rows_per_w
        pltpu.sync_copy(idx_hbm.at[pl.ds(wid * n_chunks, n_chunks)], idx_v)
        gathers = [
            pltpu.async_copy(table_hbm.at[idx_v.at[j]],
                             rows_v.at[pl.ds(j * 128, 128)], gsem)
            for j in range(n_chunks)
        ]
        writes = []
        for j in range(n_chunks):
            gathers[j].wait()
            writes.append(pltpu.async_copy(
                rows_v.at[pl.ds(j * 128, 128)],
                out_hbm.at[pl.ds(base + j * 128, 128)], wsem))
        for w in writes:
            w.wait()

    return sc_gather


def kernel(x, embedding_weight):
    B, C, H, W = x.shape
    K, D = embedding_weight.shape
    N = H * W
    NTOT = B * N

    x3 = x.reshape(B, C, N)

    idx2 = pl.pallas_call(
        _dist_argmin_body,
        grid=(B,),
        in_specs=[
            pl.BlockSpec((1, C, N), lambda b: (b, 0, 0)),
            pl.BlockSpec((K, D), lambda b: (0, 0)),
        ],
        out_specs=pl.BlockSpec((N // 128, 128), lambda b: (b, 0)),
        out_shape=jax.ShapeDtypeStruct((NTOT // 128, 128), jnp.int32),
    )(x3, embedding_weight)
    idx_flat = idx2.reshape(NTOT)

    quant = _make_sc_gather(K, D, NTOT)(embedding_weight, idx2)

    st, loss_sum = pl.pallas_call(
        _st_loss_body,
        grid=(B,),
        in_specs=[
            pl.BlockSpec((1, C, N), lambda b: (b, 0, 0)),
            pl.BlockSpec((N, C), lambda b: (b, 0)),
        ],
        out_specs=[
            pl.BlockSpec((N, C), lambda b: (b, 0)),
            pl.BlockSpec((1, 1), lambda b: (0, 0)),
        ],
        out_shape=[
            jax.ShapeDtypeStruct((NTOT, C), jnp.float32),
            jax.ShapeDtypeStruct((1, 1), jnp.float32),
        ],
    )(x3, quant)

    loss = loss_sum[0, 0] / (NTOT * C)
    quant_out = st.reshape(B, C, H, W).transpose(0, 3, 1, 2)
    idx_out = idx_flat.reshape(-1, quant_out.shape[-2], quant_out.shape[-1])
    return (quant_out, loss, loss, idx_out)


# speedup vs baseline: 1.4335x; 1.4335x over previous
import functools

import jax
import jax.numpy as jnp
from jax import lax
from jax.experimental import pallas as pl
from jax.experimental.pallas import tpu as pltpu
from jax.experimental.pallas import tpu_sc as plsc


def _dist_argmin_body(xf_ref, e_ref, idx_ref, loss_ref):
    xn = xf_ref[0]
    ew = e_ref[...]
    a2 = jnp.sum(xn * xn, axis=1, keepdims=True)
    b2 = jnp.sum(ew * ew, axis=1)[None, :]
    ab = lax.dot_general(xn, ew, (((1,), (1,)), ((), ())),
                         preferred_element_type=jnp.float32)
    d2 = jnp.maximum(a2 + b2 - 2.0 * ab, 0.0)
    d = jnp.sqrt(d2)
    idx = jnp.argmin(d, axis=1).astype(jnp.int32)
    idx_ref[...] = idx.reshape(idx_ref.shape)
    m = jnp.min(d, axis=1)

    @pl.when(pl.program_id(0) == 0)
    def _init():
        loss_ref[...] = jnp.zeros((1, 1), jnp.float32)

    loss_ref[...] += jnp.sum(m * m).reshape(1, 1)


def _make_sc_gather(K, D, NTOT):
    info = plsc.get_sparse_core_info()
    nw = info.num_cores * info.num_subcores
    rows_per_w = NTOT // nw
    n_chunks = rows_per_w // 128
    mesh = plsc.VectorSubcoreMesh(core_axis_name="c", subcore_axis_name="s")

    @functools.partial(
        pl.kernel, mesh=mesh,
        out_type=jax.ShapeDtypeStruct((NTOT, D), jnp.float32),
        scratch_types=[
            pltpu.VMEM((n_chunks, 128), jnp.int32),
            pltpu.VMEM((rows_per_w, D), jnp.float32),
            pltpu.SemaphoreType.DMA,
            pltpu.SemaphoreType.DMA,
        ],
    )
    def sc_gather(table_hbm, idx_hbm, out_hbm, idx_v, rows_v, gsem, wsem):
        wid = lax.axis_index("s") * info.num_cores + lax.axis_index("c")
        base = wid * rows_per_w
        pltpu.sync_copy(idx_hbm.at[pl.ds(wid * n_chunks, n_chunks)], idx_v)
        gathers = [
            pltpu.async_copy(table_hbm.at[idx_v.at[j]],
                             rows_v.at[pl.ds(j * 128, 128)], gsem)
            for j in range(n_chunks)
        ]
        writes = []
        for j in range(n_chunks):
            gathers[j].wait()
            writes.append(pltpu.async_copy(
                rows_v.at[pl.ds(j * 128, 128)],
                out_hbm.at[pl.ds(base + j * 128, 128)], wsem))
        for w in writes:
            w.wait()

    return sc_gather


def kernel(x, embedding_weight):
    B, C, H, W = x.shape
    K, D = embedding_weight.shape
    N = H * W
    NTOT = B * N

    xf = x.transpose(0, 2, 3, 1).reshape(NTOT, C)

    idx2, loss_sum = pl.pallas_call(
        _dist_argmin_body,
        grid=(B,),
        in_specs=[
            pl.BlockSpec((1, N, C), lambda b: (b, 0, 0)),
            pl.BlockSpec((K, D), lambda b: (0, 0)),
        ],
        out_specs=[
            pl.BlockSpec((N // 128, 128), lambda b: (b, 0)),
            pl.BlockSpec((1, 1), lambda b: (0, 0)),
        ],
        out_shape=[
            jax.ShapeDtypeStruct((NTOT // 128, 128), jnp.int32),
            jax.ShapeDtypeStruct((1, 1), jnp.float32),
        ],
    )(xf.reshape(B, N, C), embedding_weight)
    idx_flat = idx2.reshape(NTOT)

    quant = _make_sc_gather(K, D, NTOT)(embedding_weight, idx2)

    loss = loss_sum[0, 0] / (NTOT * C)
    quant_out = quant.reshape(B, C, H, W).transpose(0, 3, 1, 2)
    idx_out = idx_flat.reshape(-1, quant_out.shape[-2], quant_out.shape[-1])
    return (quant_out, loss, loss, idx_out)
